# baseline (speedup 1.0000x reference)
import functools
import math

import jax
import jax.numpy as jnp
import numpy as np
from jax import lax
from jax.experimental import pallas as pl
from jax.experimental.pallas import tpu as pltpu
from jax.experimental.pallas import tpu_sc as plsc

_D = 128
_QN = 4096
_KN = 100000
_TOPK = 64

_GRP = 128
_KNP = 102400
_NG = _KNP // _GRP
_CHUNK = 1024
_NCH = _KNP // _CHUNK
_CAPG = 192
_CAP = 256

_NEG = np.float32(-1e30)
_NEGINF = np.float32(-3e38)

_BQ = 512
_BK = 2048


def _scores_body(q_ref, k_ref, s_ref, g_ref):
    ki = pl.program_id(0)
    scale = jnp.float32(1.0 / math.sqrt(_D))
    s = lax.dot_general(q_ref[...], k_ref[...],
                        (((1,), (1,)), ((), ())),
                        preferred_element_type=jnp.float32) * scale
    gidx = ki * _BK + lax.broadcasted_iota(jnp.int32, (_BQ, _BK), 1)
    s = jnp.where(gidx < _KN, s, _NEG)
    s_ref[...] = s.reshape(_BQ, _BK // _GRP, _GRP)
    g = jnp.max(s.reshape(_BQ, _BK // _GRP, _GRP), axis=2)
    g_ref[...] = g.T


def _scores(q, kp):
    return pl.pallas_call(
        _scores_body,
        grid=(_KNP // _BK, _QN // _BQ),
        in_specs=[
            pl.BlockSpec((_BQ, _D), lambda ki, qi: (qi, 0)),
            pl.BlockSpec((_BK, _D), lambda ki, qi: (ki, 0)),
        ],
        out_specs=[
            pl.BlockSpec((_BQ, _BK // _GRP, _GRP), lambda ki, qi: (qi, ki, 0)),
            pl.BlockSpec((_BK // _GRP, _BQ), lambda ki, qi: (ki, qi)),
        ],
        out_shape=[
            jax.ShapeDtypeStruct((_QN, _NG, _GRP), jnp.float32),
            jax.ShapeDtypeStruct((_NG, _QN), jnp.float32),
        ],
    )(q, kp)


_BQ2 = 512


def _thresh_body(g_ref, t_ref):
    g = g_ref[...]
    cmax = jnp.max(g.reshape(_NCH, _CHUNK // _GRP, _BQ2), axis=1)
    c = jnp.concatenate(
        [cmax, jnp.full((128 - _NCH, _BQ2), _NEG, jnp.float32)], axis=0)
    sub = lax.broadcasted_iota(jnp.int32, (128, _BQ2), 0)

    def step(_, carry):
        c, _m = carry
        m = jnp.max(c, axis=0, keepdims=True)
        eq = c == m
        pos = jnp.min(jnp.where(eq, sub, jnp.int32(1 << 30)),
                      axis=0, keepdims=True)
        c = jnp.where(sub == pos, _NEG, c)
        return c, m

    _, m = lax.fori_loop(0, _TOPK, step, (c, jnp.zeros((1, _BQ2), jnp.float32)))
    t_ref[...] = m


def _thresholds(gmax_t):
    return pl.pallas_call(
        _thresh_body,
        grid=(_QN // _BQ2,),
        in_specs=[pl.BlockSpec((_NG, _BQ2), lambda i: (0, i))],
        out_specs=pl.BlockSpec((1, _BQ2), lambda i: (0, i)),
        out_shape=jax.ShapeDtypeStruct((1, _QN), jnp.float32),
    )(gmax_t)


_NWORKERS = 32
_ROWS_PER_W = _QN // _NWORKERS
_SLACK = _CAP + _CAPG * _GRP


def _cand_body(scores2d, gmax_hbm, thr_hbm, vals_hbm, idx_hbm,
               gmax_v, thr_v, gids_a, gids_b, grp_a, grp_b,
               cv_v, ci_v, sem):
    cid = lax.axis_index("c")
    sid = lax.axis_index("s")
    wid = sid * 2 + cid
    iota16 = lax.iota(jnp.int32, 16)
    negs = jnp.full((16,), _NEG, jnp.float32)
    zeros = jnp.zeros((16,), jnp.int32)

    pltpu.sync_copy(thr_hbm.at[wid], thr_v)

    def stage1(i, gids_v, grp_v):
        r = wid * _ROWS_PER_W + i
        rbase = r * _NG
        pltpu.sync_copy(gmax_hbm.at[r], gmax_v)
        t = plsc.load_gather(thr_v, [jnp.full((16,), i, jnp.int32)])
        dummy = jnp.full((16,), rbase + _NG - 1, jnp.int32)

        def init_g(j, _):
            gids_v[pl.ds(j * 16, 16)] = dummy
            return 0

        lax.fori_loop(0, _CAPG // 16, init_g, 0)

        def scan_body(j, ngv):
            g = gmax_v[pl.ds(j * 16, 16)]
            m = g >= t
            mi = m.astype(jnp.int32)
            pos = jnp.where(m, ngv + plsc.cumsum(mi) - 1, jnp.int32(_NG))
            plsc.store_scatter(gids_v, [pos], rbase + j * 16 + iota16)
            return ngv + plsc.all_reduce_population_count(m)

        ngv = lax.fori_loop(0, _NG // 16, scan_body,
                            jnp.zeros((16,), jnp.int32))
        ng = jnp.max(ngv)
        pltpu.async_copy(scores2d.at[gids_v.at[pl.ds(0, 96)]],
                         grp_v.at[pl.ds(0, 96)], sem)

        @pl.when(ng > 96)
        def _():
            pltpu.async_copy(scores2d.at[gids_v.at[pl.ds(96, 96)]],
                             grp_v.at[pl.ds(96, 96)], sem)

        return ng

    def stage2(i, ng, gids_v, grp_v):
        r = wid * _ROWS_PER_W + i
        rbase = r * _NG
        t = plsc.load_gather(thr_v, [jnp.full((16,), i, jnp.int32)])

        def init_c(j, _):
            cv_v[pl.ds(j * 16, 16)] = negs
            ci_v[pl.ds(j * 16, 16)] = zeros
            return 0

        lax.fori_loop(0, _CAP // 16, init_c, 0)

        pltpu.make_async_copy(scores2d.at[gids_v.at[pl.ds(0, 96)]],
                              grp_v.at[pl.ds(0, 96)], sem).wait()

        @pl.when(ng > 96)
        def _():
            pltpu.make_async_copy(scores2d.at[gids_v.at[pl.ds(96, 96)]],
                                  grp_v.at[pl.ds(96, 96)], sem).wait()

        def filt_body(jj, ncv):
            j = lax.shift_right_logical(jj, 3)
            col = (jj & 7) * 16 + iota16
            js = jnp.full((16,), j, jnp.int32)
            v = plsc.load_gather(grp_v, [js, col])
            m = v >= t

            @pl.when(jnp.any(m))
            def _():
                mi = m.astype(jnp.int32)
                pos = jnp.where(m, ncv + plsc.cumsum(mi) - 1,
                                jnp.int32(_SLACK))
                plsc.store_scatter(cv_v, [pos], v)
                plsc.store_scatter(ci_v, [pos], jj * 16 + iota16)

            return ncv + plsc.all_reduce_population_count(m)

        nlive = jnp.minimum(ng, jnp.int32(_CAPG))
        lax.fori_loop(0, nlive * 8, filt_body, jnp.zeros((16,), jnp.int32))

        def conv_body(u, _):
            a = ci_v[pl.ds(u * 16, 16)]
            gid = plsc.load_gather(gids_v, [lax.shift_right_logical(a, 7)])
            ci_v[pl.ds(u * 16, 16)] = (gid - rbase) * _GRP + (a & 127)
            return 0

        lax.fori_loop(0, _CAP // 16, conv_body, 0)

        pltpu.sync_copy(cv_v.at[pl.ds(0, _CAP)], vals_hbm.at[r])
        pltpu.sync_copy(ci_v.at[pl.ds(0, _CAP)], idx_hbm.at[r])

    def pair_body(p, ng_prev):
        a = 2 * p
        ng_a = stage1(a, gids_a, grp_a)

        @pl.when(p > 0)
        def _():
            stage2(a - 1, ng_prev, gids_b, grp_b)

        ng_b = stage1(a + 1, gids_b, grp_b)
        stage2(a, ng_a, gids_a, grp_a)
        return ng_b

    ng_last = lax.fori_loop(0, _ROWS_PER_W // 2, pair_body, jnp.int32(0))
    stage2(_ROWS_PER_W - 1, ng_last, gids_b, grp_b)


def _candidates(scores2d, gmax, thr):
    mesh = plsc.VectorSubcoreMesh(core_axis_name="c", subcore_axis_name="s")
    f = functools.partial(
        pl.kernel,
        out_type=[
            jax.ShapeDtypeStruct((_QN, _CAP), jnp.float32),
            jax.ShapeDtypeStruct((_QN, _CAP), jnp.int32),
        ],
        mesh=mesh,
        compiler_params=pltpu.CompilerParams(needs_layout_passes=False),
        scratch_types=[
            pltpu.VMEM((_NG,), jnp.float32),
            pltpu.VMEM((_ROWS_PER_W,), jnp.float32),
            pltpu.VMEM((_NG + 16,), jnp.int32),
            pltpu.VMEM((_NG + 16,), jnp.int32),
            pltpu.VMEM((_CAPG, _GRP), jnp.float32),
            pltpu.VMEM((_CAPG, _GRP), jnp.float32),
            pltpu.VMEM((_SLACK + 16,), jnp.float32),
            pltpu.VMEM((_SLACK + 16,), jnp.int32),
            pltpu.SemaphoreType.DMA,
        ],
    )(_cand_body)
    return f(scores2d, gmax, thr)


_BQ3 = 512


def _select_body(cv_ref, ci_ref, ov_ref, oi_ref):
    vals0 = cv_ref[...]
    idx = ci_ref[...]
    olane = lax.broadcasted_iota(jnp.int32, (_BQ3, _TOPK), 1)

    def step(i, carry):
        vals, ov, oi = carry
        m = jnp.max(vals, axis=1, keepdims=True)
        eq = vals == m
        ii = jnp.min(jnp.where(eq, idx, jnp.int32(1 << 30)),
                     axis=1, keepdims=True)
        hit = idx == ii
        sel = olane == i
        ov = jnp.where(sel, m, ov)
        oi = jnp.where(sel, ii, oi)
        return jnp.where(hit, _NEGINF, vals), ov, oi

    _, ov, oi = lax.fori_loop(
        0, _TOPK, step,
        (vals0, jnp.zeros((_BQ3, _TOPK), jnp.float32),
         jnp.zeros((_BQ3, _TOPK), jnp.int32)))
    ov_ref[...] = ov
    oi_ref[...] = oi


def _select(cv, ci):
    return pl.pallas_call(
        _select_body,
        grid=(_QN // _BQ3,),
        in_specs=[
            pl.BlockSpec((_BQ3, _CAP), lambda i: (i, 0)),
            pl.BlockSpec((_BQ3, _CAP), lambda i: (i, 0)),
        ],
        out_specs=[
            pl.BlockSpec((_BQ3, _TOPK), lambda i: (i, 0)),
            pl.BlockSpec((_BQ3, _TOPK), lambda i: (i, 0)),
        ],
        out_shape=[
            jax.ShapeDtypeStruct((_QN, _TOPK), jnp.float32),
            jax.ShapeDtypeStruct((_QN, _TOPK), jnp.int32),
        ],
    )(cv, ci)


def kernel(queries, keys, k):
    kp = jnp.concatenate(
        [keys, jnp.zeros((_KNP - _KN, _D), jnp.float32)], axis=0)
    scores, gmax_t = _scores(queries, kp)
    thr = _thresholds(gmax_t)
    scores2d = scores.reshape(_QN * _NG, _GRP)
    cv, ci = _candidates(scores2d, gmax_t.T,
                         thr.reshape(_NWORKERS, _ROWS_PER_W))
    return tuple(_select(cv, ci))


# speedup vs baseline: 1.5165x; 1.5165x over previous
import functools
import math

import jax
import jax.numpy as jnp
import numpy as np
from jax import lax
from jax.experimental import pallas as pl
from jax.experimental.pallas import tpu as pltpu
from jax.experimental.pallas import tpu_sc as plsc

_D = 128
_QN = 4096
_KN = 100000
_TOPK = 64

_GRP = 128
_KNP = 102400
_NG = _KNP // _GRP
_CHUNK = 1024
_NCH = _KNP // _CHUNK
_CAPG = 192
_CAP = 256

_NEG = np.float32(-1e30)
_NEGINF = np.float32(-3e38)

_BQ = 512
_BK = 2048


def _scores_body(q_ref, k_ref, s_ref, g_ref):
    ki = pl.program_id(0)
    scale = jnp.float32(1.0 / math.sqrt(_D))
    s = lax.dot_general(q_ref[...], k_ref[...],
                        (((1,), (1,)), ((), ())),
                        preferred_element_type=jnp.float32) * scale
    gidx = ki * _BK + lax.broadcasted_iota(jnp.int32, (_BQ, _BK), 1)
    s = jnp.where(gidx < _KN, s, _NEG)
    s_ref[...] = s.reshape(_BQ, _BK // _GRP, _GRP)
    g = jnp.max(s.reshape(_BQ, _BK // _GRP, _GRP), axis=2)
    g_ref[...] = g.T


def _scores(q, kp):
    return pl.pallas_call(
        _scores_body,
        grid=(_KNP // _BK, _QN // _BQ),
        in_specs=[
            pl.BlockSpec((_BQ, _D), lambda ki, qi: (qi, 0)),
            pl.BlockSpec((_BK, _D), lambda ki, qi: (ki, 0)),
        ],
        out_specs=[
            pl.BlockSpec((_BQ, _BK // _GRP, _GRP), lambda ki, qi: (qi, ki, 0)),
            pl.BlockSpec((_BK // _GRP, _BQ), lambda ki, qi: (ki, qi)),
        ],
        out_shape=[
            jax.ShapeDtypeStruct((_QN, _NG, _GRP), jnp.float32),
            jax.ShapeDtypeStruct((_NG, _QN), jnp.float32),
        ],
    )(q, kp)


_BQ2 = 512


def _thresh_body(g_ref, t_ref):
    g = g_ref[...]
    cmax = jnp.max(g.reshape(_NCH, _CHUNK // _GRP, _BQ2), axis=1)
    c = jnp.concatenate(
        [cmax, jnp.full((128 - _NCH, _BQ2), _NEG, jnp.float32)], axis=0)
    sub = lax.broadcasted_iota(jnp.int32, (128, _BQ2), 0)

    def step(_, carry):
        c, _m = carry
        m = jnp.max(c, axis=0, keepdims=True)
        eq = c == m
        pos = jnp.min(jnp.where(eq, sub, jnp.int32(1 << 30)),
                      axis=0, keepdims=True)
        c = jnp.where(sub == pos, _NEG, c)
        return c, m

    _, m = lax.fori_loop(0, _TOPK, step, (c, jnp.zeros((1, _BQ2), jnp.float32)))
    t_ref[...] = m


def _thresholds(gmax_t):
    return pl.pallas_call(
        _thresh_body,
        grid=(_QN // _BQ2,),
        in_specs=[pl.BlockSpec((_NG, _BQ2), lambda i: (0, i))],
        out_specs=pl.BlockSpec((1, _BQ2), lambda i: (0, i)),
        out_shape=jax.ShapeDtypeStruct((1, _QN), jnp.float32),
    )(gmax_t)


_NWORKERS = 32
_ROWS_PER_W = _QN // _NWORKERS
_SLACK = _CAP + _CAPG * _GRP


def _cand_body(scores2d, gmax_hbm, thr_hbm, vals_hbm, idx_hbm,
               gmax_v, thr_v, gids_a, gids_b, grp_a, grp_b,
               cv_v, ci_v, sem):
    cid = lax.axis_index("c")
    sid = lax.axis_index("s")
    wid = sid * 2 + cid
    iota16 = lax.iota(jnp.int32, 16)
    negs = jnp.full((16,), _NEG, jnp.float32)
    zeros = jnp.zeros((16,), jnp.int32)

    pltpu.sync_copy(thr_hbm.at[wid], thr_v)

    def stage1(i, gids_v, grp_v):
        r = wid * _ROWS_PER_W + i
        rbase = r * _NG
        pltpu.sync_copy(gmax_hbm.at[r], gmax_v)
        t = plsc.load_gather(thr_v, [jnp.full((16,), i, jnp.int32)])
        dummy = jnp.full((16,), rbase + _NG - 1, jnp.int32)

        def init_g(j, _):
            gids_v[pl.ds(j * 16, 16)] = dummy
            return 0

        lax.fori_loop(0, _CAPG // 16, init_g, 0)

        def scan_body(j, ngv):
            g = gmax_v[pl.ds(j * 16, 16)]
            m = g >= t
            mi = m.astype(jnp.int32)
            pos = jnp.where(m, ngv + plsc.cumsum(mi) - 1, jnp.int32(_NG))
            plsc.store_scatter(gids_v, [pos], rbase + j * 16 + iota16)
            return ngv + plsc.all_reduce_population_count(m)

        ngv = lax.fori_loop(0, _NG // 16, scan_body,
                            jnp.zeros((16,), jnp.int32))
        ng = jnp.max(ngv)
        pltpu.async_copy(scores2d.at[gids_v.at[pl.ds(0, 96)]],
                         grp_v.at[pl.ds(0, 96)], sem)

        @pl.when(ng > 96)
        def _():
            pltpu.async_copy(scores2d.at[gids_v.at[pl.ds(96, 96)]],
                             grp_v.at[pl.ds(96, 96)], sem)

        return ng

    def stage2(i, ng, gids_v, grp_v):
        r = wid * _ROWS_PER_W + i
        rbase = r * _NG
        t = plsc.load_gather(thr_v, [jnp.full((16,), i, jnp.int32)])

        def init_c(j, _):
            cv_v[pl.ds(j * 16, 16)] = negs
            ci_v[pl.ds(j * 16, 16)] = zeros
            return 0

        lax.fori_loop(0, _CAP // 16, init_c, 0)

        pltpu.make_async_copy(scores2d.at[gids_v.at[pl.ds(0, 96)]],
                              grp_v.at[pl.ds(0, 96)], sem).wait()

        @pl.when(ng > 96)
        def _():
            pltpu.make_async_copy(scores2d.at[gids_v.at[pl.ds(96, 96)]],
                                  grp_v.at[pl.ds(96, 96)], sem).wait()

        def filt_body(jj, ncv):
            j = lax.shift_right_logical(jj, 3)
            col = (jj & 7) * 16 + iota16
            js = jnp.full((16,), j, jnp.int32)
            v = plsc.load_gather(grp_v, [js, col])
            m = v >= t
            mi = m.astype(jnp.int32)
            pos = jnp.where(m, ncv + plsc.cumsum(mi) - 1, jnp.int32(_SLACK))
            plsc.store_scatter(cv_v, [pos], v)
            plsc.store_scatter(ci_v, [pos], jj * 16 + iota16)
            return ncv + plsc.all_reduce_population_count(m)

        nlive = jnp.minimum(ng, jnp.int32(_CAPG))
        lax.fori_loop(0, nlive * 8, filt_body, jnp.zeros((16,), jnp.int32))

        def conv_body(u, _):
            a = ci_v[pl.ds(u * 16, 16)]
            gid = plsc.load_gather(gids_v, [lax.shift_right_logical(a, 7)])
            ci_v[pl.ds(u * 16, 16)] = (gid - rbase) * _GRP + (a & 127)
            return 0

        lax.fori_loop(0, _CAP // 16, conv_body, 0)

        pltpu.sync_copy(cv_v.at[pl.ds(0, _CAP)], vals_hbm.at[r])
        pltpu.sync_copy(ci_v.at[pl.ds(0, _CAP)], idx_hbm.at[r])

    def pair_body(p, ng_prev):
        a = 2 * p
        ng_a = stage1(a, gids_a, grp_a)

        @pl.when(p > 0)
        def _():
            stage2(a - 1, ng_prev, gids_b, grp_b)

        ng_b = stage1(a + 1, gids_b, grp_b)
        stage2(a, ng_a, gids_a, grp_a)
        return ng_b

    ng_last = lax.fori_loop(0, _ROWS_PER_W // 2, pair_body, jnp.int32(0))
    stage2(_ROWS_PER_W - 1, ng_last, gids_b, grp_b)


def _candidates(scores2d, gmax, thr):
    mesh = plsc.VectorSubcoreMesh(core_axis_name="c", subcore_axis_name="s")
    f = functools.partial(
        pl.kernel,
        out_type=[
            jax.ShapeDtypeStruct((_QN, _CAP), jnp.float32),
            jax.ShapeDtypeStruct((_QN, _CAP), jnp.int32),
        ],
        mesh=mesh,
        compiler_params=pltpu.CompilerParams(needs_layout_passes=False),
        scratch_types=[
            pltpu.VMEM((_NG,), jnp.float32),
            pltpu.VMEM((_ROWS_PER_W,), jnp.float32),
            pltpu.VMEM((_NG + 16,), jnp.int32),
            pltpu.VMEM((_NG + 16,), jnp.int32),
            pltpu.VMEM((_CAPG, _GRP), jnp.float32),
            pltpu.VMEM((_CAPG, _GRP), jnp.float32),
            pltpu.VMEM((_SLACK + 16,), jnp.float32),
            pltpu.VMEM((_SLACK + 16,), jnp.int32),
            pltpu.SemaphoreType.DMA,
        ],
    )(_cand_body)
    return f(scores2d, gmax, thr)


_BQ3 = 512


def _select_body(cv_ref, ci_ref, ov_ref, oi_ref):
    vals0 = cv_ref[...]
    idx = ci_ref[...]
    olane = lax.broadcasted_iota(jnp.int32, (_BQ3, _TOPK), 1)

    def step(i, carry):
        vals, ov, oi = carry
        m = jnp.max(vals, axis=1, keepdims=True)
        eq = vals == m
        ii = jnp.min(jnp.where(eq, idx, jnp.int32(1 << 30)),
                     axis=1, keepdims=True)
        hit = idx == ii
        sel = olane == i
        ov = jnp.where(sel, m, ov)
        oi = jnp.where(sel, ii, oi)
        return jnp.where(hit, _NEGINF, vals), ov, oi

    _, ov, oi = lax.fori_loop(
        0, _TOPK, step,
        (vals0, jnp.zeros((_BQ3, _TOPK), jnp.float32),
         jnp.zeros((_BQ3, _TOPK), jnp.int32)))
    ov_ref[...] = ov
    oi_ref[...] = oi


def _select(cv, ci):
    return pl.pallas_call(
        _select_body,
        grid=(_QN // _BQ3,),
        in_specs=[
            pl.BlockSpec((_BQ3, _CAP), lambda i: (i, 0)),
            pl.BlockSpec((_BQ3, _CAP), lambda i: (i, 0)),
        ],
        out_specs=[
            pl.BlockSpec((_BQ3, _TOPK), lambda i: (i, 0)),
            pl.BlockSpec((_BQ3, _TOPK), lambda i: (i, 0)),
        ],
        out_shape=[
            jax.ShapeDtypeStruct((_QN, _TOPK), jnp.float32),
            jax.ShapeDtypeStruct((_QN, _TOPK), jnp.int32),
        ],
    )(cv, ci)


def kernel(queries, keys, k):
    kp = jnp.concatenate(
        [keys, jnp.zeros((_KNP - _KN, _D), jnp.float32)], axis=0)
    scores, gmax_t = _scores(queries, kp)
    thr = _thresholds(gmax_t)
    scores2d = scores.reshape(_QN * _NG, _GRP)
    cv, ci = _candidates(scores2d, gmax_t.T,
                         thr.reshape(_NWORKERS, _ROWS_PER_W))
    return tuple(_select(cv, ci))


# speedup vs baseline: 1.5758x; 1.0391x over previous
import functools
import math

import jax
import jax.numpy as jnp
import numpy as np
from jax import lax
from jax.experimental import pallas as pl
from jax.experimental.pallas import tpu as pltpu
from jax.experimental.pallas import tpu_sc as plsc

_D = 128
_QN = 4096
_KN = 100000
_TOPK = 64

_GRP = 128
_KNP = 102400
_NG = _KNP // _GRP
_CHUNK = 1024
_NCH = _KNP // _CHUNK
_CAPG = 192
_CAP = 256

_NEG = np.float32(-1e30)
_NEGINF = np.float32(-3e38)

_BQ = 512
_BK = 2048


def _scores_body(q_ref, k_ref, s_ref, g_ref):
    ki = pl.program_id(0)
    scale = jnp.float32(1.0 / math.sqrt(_D))
    s = lax.dot_general(q_ref[...], k_ref[...],
                        (((1,), (1,)), ((), ())),
                        preferred_element_type=jnp.float32) * scale
    gidx = ki * _BK + lax.broadcasted_iota(jnp.int32, (_BQ, _BK), 1)
    s = jnp.where(gidx < _KN, s, _NEG)
    s_ref[...] = s.reshape(_BQ, _BK // _GRP, _GRP)
    g = jnp.max(s.reshape(_BQ, _BK // _GRP, _GRP), axis=2)
    g_ref[...] = g.T


def _scores(q, kp):
    return pl.pallas_call(
        _scores_body,
        grid=(_KNP // _BK, _QN // _BQ),
        in_specs=[
            pl.BlockSpec((_BQ, _D), lambda ki, qi: (qi, 0)),
            pl.BlockSpec((_BK, _D), lambda ki, qi: (ki, 0)),
        ],
        out_specs=[
            pl.BlockSpec((_BQ, _BK // _GRP, _GRP), lambda ki, qi: (qi, ki, 0)),
            pl.BlockSpec((_BK // _GRP, _BQ), lambda ki, qi: (ki, qi)),
        ],
        out_shape=[
            jax.ShapeDtypeStruct((_QN, _NG, _GRP), jnp.float32),
            jax.ShapeDtypeStruct((_NG, _QN), jnp.float32),
        ],
    )(q, kp)


_BQ2 = 512


def _thresh_body(g_ref, t_ref):
    g = g_ref[...]
    cmax = jnp.max(g.reshape(_NCH, _CHUNK // _GRP, _BQ2), axis=1)
    c = jnp.concatenate(
        [cmax, jnp.full((128 - _NCH, _BQ2), _NEG, jnp.float32)], axis=0)
    sub = lax.broadcasted_iota(jnp.int32, (128, _BQ2), 0)

    def step(_, carry):
        c, _m = carry
        m = jnp.max(c, axis=0, keepdims=True)
        eq = c == m
        pos = jnp.min(jnp.where(eq, sub, jnp.int32(1 << 30)),
                      axis=0, keepdims=True)
        c = jnp.where(sub == pos, _NEG, c)
        return c, m

    _, m = lax.fori_loop(0, _TOPK, step, (c, jnp.zeros((1, _BQ2), jnp.float32)))
    t_ref[...] = m


def _thresholds(gmax_t):
    return pl.pallas_call(
        _thresh_body,
        grid=(_QN // _BQ2,),
        in_specs=[pl.BlockSpec((_NG, _BQ2), lambda i: (0, i))],
        out_specs=pl.BlockSpec((1, _BQ2), lambda i: (0, i)),
        out_shape=jax.ShapeDtypeStruct((1, _QN), jnp.float32),
    )(gmax_t)


_NWORKERS = 32
_ROWS_PER_W = _QN // _NWORKERS
_SLACK = _CAP + _CAPG * _GRP


def _cand_body(scores2d, gmax_hbm, thr_hbm, vals_hbm, idx_hbm,
               gmax_v, thr_v, gids_a, gids_b, grp_a, grp_b,
               cv_v, ci_v, sem):
    cid = lax.axis_index("c")
    sid = lax.axis_index("s")
    wid = sid * 2 + cid
    iota16 = lax.iota(jnp.int32, 16)
    negs = jnp.full((16,), _NEG, jnp.float32)
    zeros = jnp.zeros((16,), jnp.int32)

    pltpu.sync_copy(thr_hbm.at[wid], thr_v)

    def stage1(i, gids_v, grp_v):
        r = wid * _ROWS_PER_W + i
        rbase = r * _NG
        pltpu.sync_copy(gmax_hbm.at[r], gmax_v)
        t = plsc.load_gather(thr_v, [jnp.full((16,), i, jnp.int32)])
        dummy = jnp.full((16,), rbase + _NG - 1, jnp.int32)

        def init_g(j, _):
            gids_v[pl.ds(j * 16, 16)] = dummy
            return 0

        lax.fori_loop(0, _CAPG // 16, init_g, 0)

        def scan_body(j, ngv):
            g = gmax_v[pl.ds(j * 16, 16)]
            m = g >= t
            mi = m.astype(jnp.int32)
            pos = jnp.where(m, ngv + plsc.cumsum(mi) - 1, jnp.int32(_NG))
            plsc.store_scatter(gids_v, [pos], rbase + j * 16 + iota16)
            return ngv + plsc.all_reduce_population_count(m)

        ngv = lax.fori_loop(0, _NG // 16, scan_body,
                            jnp.zeros((16,), jnp.int32))
        ng = jnp.max(ngv)
        pltpu.async_copy(scores2d.at[gids_v.at[pl.ds(0, 96)]],
                         grp_v.at[pl.ds(0, 96)], sem)

        @pl.when(ng > 96)
        def _():
            pltpu.async_copy(scores2d.at[gids_v.at[pl.ds(96, 96)]],
                             grp_v.at[pl.ds(96, 96)], sem)

        return ng

    def stage2(i, ng, gids_v, grp_v):
        r = wid * _ROWS_PER_W + i
        rbase = r * _NG
        t = plsc.load_gather(thr_v, [jnp.full((16,), i, jnp.int32)])

        def init_c(j, _):
            cv_v[pl.ds(j * 16, 16)] = negs
            ci_v[pl.ds(j * 16, 16)] = zeros
            return 0

        lax.fori_loop(0, _CAP // 16, init_c, 0)

        pltpu.make_async_copy(scores2d.at[gids_v.at[pl.ds(0, 96)]],
                              grp_v.at[pl.ds(0, 96)], sem).wait()

        @pl.when(ng > 96)
        def _():
            pltpu.make_async_copy(scores2d.at[gids_v.at[pl.ds(96, 96)]],
                                  grp_v.at[pl.ds(96, 96)], sem).wait()

        def filt_body(jj, ncv):
            j = lax.shift_right_logical(jj, 3)
            c0 = (jj & 7) * 16
            v = grp_v[j, pl.ds(c0, 16)]
            m = v >= t
            mi = m.astype(jnp.int32)
            pos = jnp.where(m, ncv + plsc.cumsum(mi) - 1, jnp.int32(_SLACK))
            plsc.store_scatter(cv_v, [pos], v)
            plsc.store_scatter(ci_v, [pos], jj * 16 + iota16)
            return ncv + plsc.all_reduce_population_count(m)

        nlive = jnp.minimum(ng, jnp.int32(_CAPG))
        lax.fori_loop(0, nlive * 8, filt_body, jnp.zeros((16,), jnp.int32))

        def conv_body(u, _):
            a = ci_v[pl.ds(u * 16, 16)]
            gid = plsc.load_gather(gids_v, [lax.shift_right_logical(a, 7)])
            ci_v[pl.ds(u * 16, 16)] = (gid - rbase) * _GRP + (a & 127)
            return 0

        lax.fori_loop(0, _CAP // 16, conv_body, 0)

        pltpu.sync_copy(cv_v.at[pl.ds(0, _CAP)], vals_hbm.at[r])
        pltpu.sync_copy(ci_v.at[pl.ds(0, _CAP)], idx_hbm.at[r])

    def pair_body(p, ng_prev):
        a = 2 * p
        ng_a = stage1(a, gids_a, grp_a)

        @pl.when(p > 0)
        def _():
            stage2(a - 1, ng_prev, gids_b, grp_b)

        ng_b = stage1(a + 1, gids_b, grp_b)
        stage2(a, ng_a, gids_a, grp_a)
        return ng_b

    ng_last = lax.fori_loop(0, _ROWS_PER_W // 2, pair_body, jnp.int32(0))
    stage2(_ROWS_PER_W - 1, ng_last, gids_b, grp_b)


def _candidates(scores2d, gmax, thr):
    mesh = plsc.VectorSubcoreMesh(core_axis_name="c", subcore_axis_name="s")
    f = functools.partial(
        pl.kernel,
        out_type=[
            jax.ShapeDtypeStruct((_QN, _CAP), jnp.float32),
            jax.ShapeDtypeStruct((_QN, _CAP), jnp.int32),
        ],
        mesh=mesh,
        compiler_params=pltpu.CompilerParams(needs_layout_passes=False),
        scratch_types=[
            pltpu.VMEM((_NG,), jnp.float32),
            pltpu.VMEM((_ROWS_PER_W,), jnp.float32),
            pltpu.VMEM((_NG + 16,), jnp.int32),
            pltpu.VMEM((_NG + 16,), jnp.int32),
            pltpu.VMEM((_CAPG, _GRP), jnp.float32),
            pltpu.VMEM((_CAPG, _GRP), jnp.float32),
            pltpu.VMEM((_SLACK + 16,), jnp.float32),
            pltpu.VMEM((_SLACK + 16,), jnp.int32),
            pltpu.SemaphoreType.DMA,
        ],
    )(_cand_body)
    return f(scores2d, gmax, thr)


_BQ3 = 512


def _select_body(cv_ref, ci_ref, ov_ref, oi_ref):
    vals0 = cv_ref[...]
    idx = ci_ref[...]
    olane = lax.broadcasted_iota(jnp.int32, (_BQ3, _TOPK), 1)

    def step(i, carry):
        vals, ov, oi = carry
        m = jnp.max(vals, axis=1, keepdims=True)
        eq = vals == m
        ii = jnp.min(jnp.where(eq, idx, jnp.int32(1 << 30)),
                     axis=1, keepdims=True)
        hit = idx == ii
        sel = olane == i
        ov = jnp.where(sel, m, ov)
        oi = jnp.where(sel, ii, oi)
        return jnp.where(hit, _NEGINF, vals), ov, oi

    _, ov, oi = lax.fori_loop(
        0, _TOPK, step,
        (vals0, jnp.zeros((_BQ3, _TOPK), jnp.float32),
         jnp.zeros((_BQ3, _TOPK), jnp.int32)))
    ov_ref[...] = ov
    oi_ref[...] = oi


def _select(cv, ci):
    return pl.pallas_call(
        _select_body,
        grid=(_QN // _BQ3,),
        in_specs=[
            pl.BlockSpec((_BQ3, _CAP), lambda i: (i, 0)),
            pl.BlockSpec((_BQ3, _CAP), lambda i: (i, 0)),
        ],
        out_specs=[
            pl.BlockSpec((_BQ3, _TOPK), lambda i: (i, 0)),
            pl.BlockSpec((_BQ3, _TOPK), lambda i: (i, 0)),
        ],
        out_shape=[
            jax.ShapeDtypeStruct((_QN, _TOPK), jnp.float32),
            jax.ShapeDtypeStruct((_QN, _TOPK), jnp.int32),
        ],
    )(cv, ci)


def kernel(queries, keys, k):
    kp = jnp.concatenate(
        [keys, jnp.zeros((_KNP - _KN, _D), jnp.float32)], axis=0)
    scores, gmax_t = _scores(queries, kp)
    thr = _thresholds(gmax_t)
    scores2d = scores.reshape(_QN * _NG, _GRP)
    cv, ci = _candidates(scores2d, gmax_t.T,
                         thr.reshape(_NWORKERS, _ROWS_PER_W))
    return tuple(_select(cv, ci))
